# Initial kernel scaffold; baseline (speedup 1.0000x reference)
import jax
import jax.numpy as jnp
from jax.experimental import pallas as pl


def kernel(inputs, edge, weights, a):
    raise NotImplementedError("write your pallas kernel here")


# speedup vs baseline: 1.1698x; 1.1698x over previous
import functools

import jax
import jax.numpy as jnp
from jax import lax
from jax.experimental import pallas as pl
from jax.experimental.pallas import tpu as pltpu
from jax.experimental.pallas import tpu_sc as plsc

ALPHA = 0.5

NC = 2
NS = 16
NW = NC * NS
CHUNK = 128


def _sc_mesh():
  return plsc.VectorSubcoreMesh(core_axis_name="c", subcore_axis_name="s")


def _worker_id():
  return lax.axis_index("s") * NC + lax.axis_index("c")


def _matmul(x, w):
  n, d_in = x.shape
  d_out = w.shape[1]
  blk = 1000
  grid = n // blk

  def body(x_ref, w_ref, o_ref):
    o_ref[...] = lax.dot_general(
        x_ref[...], w_ref[...], (((1,), (0,)), ((), ())),
        precision=lax.Precision.HIGHEST,
        preferred_element_type=jnp.float32)

  return pl.pallas_call(
      body,
      grid=(grid,),
      in_specs=[
          pl.BlockSpec((blk, d_in), lambda i: (i, 0)),
          pl.BlockSpec((d_in, d_out), lambda i: (0, 0)),
      ],
      out_specs=pl.BlockSpec((blk, d_out), lambda i: (i, 0)),
      out_shape=jax.ShapeDtypeStruct((n, d_out), jnp.float32),
  )(x, w)


def _make_k2(n, e):
  nchunk = e // CHUNK

  @functools.partial(
      pl.kernel,
      mesh=_sc_mesh(),
      compiler_params=pltpu.CompilerParams(needs_layout_passes=False),
      out_type=jax.ShapeDtypeStruct((n * n,), jnp.int32),
      scratch_types=[
          pltpu.VMEM((CHUNK,), jnp.int32),
          pltpu.VMEM((CHUNK,), jnp.int32),
          pltpu.VMEM((CHUNK,), jnp.int32),
          pltpu.VMEM((CHUNK,), jnp.int32),
          pltpu.SemaphoreType.DMA,
      ],
  )
  def k2(src_hbm, dst_hbm, s_hbm, src_v, dst_v, key_v, id_v, sem):
    wid = _worker_id()

    @pl.loop(wid, nchunk, step=NW)
    def _chunk(c):
      eb = c * CHUNK
      pltpu.sync_copy(src_hbm.at[pl.ds(eb, CHUNK)], src_v)
      pltpu.sync_copy(dst_hbm.at[pl.ds(eb, CHUNK)], dst_v)
      for j in range(CHUNK // 16):
        sl = pl.ds(j * 16, 16)
        key_v[sl] = src_v[sl] * n + dst_v[sl]
        id_v[sl] = lax.iota(jnp.int32, 16) + (eb + j * 16)
      pltpu.async_copy(id_v, s_hbm.at[key_v], sem).wait()

  return k2


def _make_k3(n, e, d):
  nchunk = e // CHUNK
  per_sub = e // NS
  qn = d // 16

  @functools.partial(
      pl.kernel,
      mesh=_sc_mesh(),
      compiler_params=pltpu.CompilerParams(needs_layout_passes=False),
      out_type=(
          jax.ShapeDtypeStruct((e,), jnp.int32),
          jax.ShapeDtypeStruct((2 * e,), jnp.float32),
          jax.ShapeDtypeStruct((2 * e,), jnp.float32),
      ),
      scratch_types=[
          pltpu.VMEM((CHUNK,), jnp.int32),
          pltpu.VMEM((CHUNK,), jnp.int32),
          pltpu.VMEM((CHUNK,), jnp.int32),
          pltpu.VMEM((CHUNK,), jnp.int32),
          pltpu.VMEM((CHUNK, 256), jnp.float32),
          pltpu.VMEM((CHUNK, 256), jnp.float32),
          pltpu.VMEM((CHUNK,), jnp.float32),
          pltpu.VMEM((CHUNK,), jnp.float32),
          pltpu.VMEM((256,), jnp.float32),
          pltpu.VMEM((2000,), jnp.float32),
          pltpu.VMEM_SHARED((e,), jnp.float32),
          pltpu.VMEM_SHARED((e,), jnp.float32),
          pltpu.SemaphoreType.DMA,
      ],
  )
  def k3(h_hbm, src_hbm, dst_hbm, a_hbm, s_hbm,
         w_out, sv_out, sc_out,
         src_v, dst_v, key_v, w_v, hs_v, hd_v, val_v, ones_v, a_v, tmp_v,
         sv_sh, sc_sh, sem):
    cid = lax.axis_index("c")
    sid = lax.axis_index("s")
    wid = sid * NC + cid

    pltpu.sync_copy(a_hbm, a_v)
    for j in range(CHUNK // 16):
      ones_v[pl.ds(j * 16, 16)] = jnp.full((16,), 1.0, jnp.float32)

    @pl.loop(0, 2000 // 16)
    def _z(i):
      tmp_v[pl.ds(i * 16, 16)] = jnp.zeros((16,), jnp.float32)

    for t in range(per_sub // 2000):
      sl = pl.ds(sid * per_sub + t * 2000, 2000)
      pltpu.sync_copy(tmp_v, sv_sh.at[sl])
      pltpu.sync_copy(tmp_v, sc_sh.at[sl])
    plsc.subcore_barrier()

    @pl.loop(wid, nchunk, step=NW)
    def _chunk(c):
      eb = c * CHUNK
      pltpu.sync_copy(src_hbm.at[pl.ds(eb, CHUNK)], src_v)
      pltpu.sync_copy(dst_hbm.at[pl.ds(eb, CHUNK)], dst_v)
      for j in range(CHUNK // 16):
        sl = pl.ds(j * 16, 16)
        key_v[sl] = src_v[sl] * n + dst_v[sl]
      cp_w = pltpu.async_copy(s_hbm.at[key_v], w_v, sem)
      cp_s = pltpu.async_copy(h_hbm.at[src_v], hs_v, sem)
      cp_d = pltpu.async_copy(h_hbm.at[dst_v], hd_v, sem)
      cp_w.wait()
      cp_s.wait()
      cp_d.wait()

      @pl.loop(0, CHUNK // 16)
      def _grp(g):
        e16 = lax.iota(jnp.int32, 16) + g * 16
        acc = jnp.zeros((16,), jnp.float32)
        for q in range(qn):
          a16 = a_v[pl.ds(q * 16, 16)]
          for t in range(16):
            col = jnp.full((16,), q * 16 + t, jnp.int32)
            hs = plsc.load_gather(hs_v, [e16, col])
            hd = plsc.load_gather(hd_v, [e16, col])
            acc = acc + jnp.abs(hs - hd) * a16[t]
        val_v[pl.ds(g * 16, 16)] = jnp.maximum(acc, 0.0)

      pltpu.sync_copy(val_v, sv_sh.at[w_v], add=True)
      pltpu.sync_copy(ones_v, sc_sh.at[w_v], add=True)
      pltpu.sync_copy(w_v, w_out.at[pl.ds(eb, CHUNK)])

    plsc.subcore_barrier()
    for t in range(per_sub // 2000):
      off = sid * per_sub + t * 2000
      sl = pl.ds(off, 2000)
      slo = pl.ds(cid * e + off, 2000)
      pltpu.sync_copy(sv_sh.at[sl], tmp_v)
      pltpu.sync_copy(tmp_v, sv_out.at[slo])
      pltpu.sync_copy(sc_sh.at[sl], tmp_v)
      pltpu.sync_copy(tmp_v, sc_out.at[slo])

  return k3


def _make_k4(n, e, npad):
  nchunk = e // CHUNK
  zslice = npad // NS

  @functools.partial(
      pl.kernel,
      mesh=_sc_mesh(),
      compiler_params=pltpu.CompilerParams(needs_layout_passes=False),
      out_type=(
          jax.ShapeDtypeStruct((2 * npad,), jnp.float32),
          jax.ShapeDtypeStruct((e,), jnp.float32),
          jax.ShapeDtypeStruct((e,), jnp.float32),
      ),
      scratch_types=[
          pltpu.VMEM((CHUNK,), jnp.int32),
          pltpu.VMEM((CHUNK,), jnp.int32),
          pltpu.VMEM((CHUNK,), jnp.float32),
          pltpu.VMEM((CHUNK,), jnp.float32),
          pltpu.VMEM((CHUNK,), jnp.float32),
          pltpu.VMEM((CHUNK,), jnp.float32),
          pltpu.VMEM((CHUNK,), jnp.float32),
          pltpu.VMEM((CHUNK,), jnp.float32),
          pltpu.VMEM((CHUNK,), jnp.float32),
          pltpu.VMEM((zslice,), jnp.float32),
          pltpu.VMEM_SHARED((npad,), jnp.float32),
          pltpu.SemaphoreType.DMA,
      ],
  )
  def k4(w_hbm, src_hbm, sv0_hbm, sv1_hbm, sc0_hbm, sc1_hbm,
         z_out, expv_out, c_out,
         w_v, src_v, t0_v, t1_v, t2_v, t3_v, expv_v, c_v, contrib_v,
         tmp_v, z_sh, sem):
    cid = lax.axis_index("c")
    sid = lax.axis_index("s")
    wid = sid * NC + cid

    @pl.loop(0, zslice // 16)
    def _z(i):
      tmp_v[pl.ds(i * 16, 16)] = jnp.zeros((16,), jnp.float32)

    pltpu.sync_copy(tmp_v, z_sh.at[pl.ds(sid * zslice, zslice)])
    plsc.subcore_barrier()

    @pl.loop(wid, nchunk, step=NW)
    def _chunk(c):
      eb = c * CHUNK
      pltpu.sync_copy(w_hbm.at[pl.ds(eb, CHUNK)], w_v)
      pltpu.sync_copy(src_hbm.at[pl.ds(eb, CHUNK)], src_v)
      cps = [
          pltpu.async_copy(sv0_hbm.at[w_v], t0_v, sem),
          pltpu.async_copy(sv1_hbm.at[w_v], t1_v, sem),
          pltpu.async_copy(sc0_hbm.at[w_v], t2_v, sem),
          pltpu.async_copy(sc1_hbm.at[w_v], t3_v, sem),
      ]
      for cp in cps:
        cp.wait()
      for j in range(CHUNK // 16):
        sl = pl.ds(j * 16, 16)
        val = t0_v[sl] + t1_v[sl]
        ex = jnp.exp(val)
        expv_v[sl] = ex
        c_v[sl] = t2_v[sl] + t3_v[sl]
        eid = lax.iota(jnp.int32, 16) + (eb + j * 16)
        contrib_v[sl] = jnp.where(w_v[sl] == eid, ex - 1.0, 0.0)
      pltpu.sync_copy(contrib_v, z_sh.at[src_v], add=True)
      pltpu.sync_copy(expv_v, expv_out.at[pl.ds(eb, CHUNK)])
      pltpu.sync_copy(c_v, c_out.at[pl.ds(eb, CHUNK)])

    plsc.subcore_barrier()
    pltpu.sync_copy(z_sh.at[pl.ds(sid * zslice, zslice)], tmp_v)
    pltpu.sync_copy(tmp_v, z_out.at[pl.ds(cid * npad + sid * zslice, zslice)])

  return k4


def _fill(z0, z1, n):
  blk = 200
  grid = n // blk
  nf = float(n)

  def body(z0_ref, z1_ref, o_ref):
    z = nf + z0_ref[...] + z1_ref[...]
    o_ref[...] = jnp.broadcast_to(1.0 / z, (blk, n))

  return pl.pallas_call(
      body,
      grid=(grid,),
      in_specs=[
          pl.BlockSpec((blk, 1), lambda i: (i, 0)),
          pl.BlockSpec((blk, 1), lambda i: (i, 0)),
      ],
      out_specs=pl.BlockSpec((blk, n), lambda i: (i, 0)),
      out_shape=jax.ShapeDtypeStruct((n, n), jnp.float32),
  )(z0, z1)


def _make_k6(n, e):
  nchunk = e // CHUNK
  nf = float(n)

  @functools.partial(
      pl.kernel,
      mesh=_sc_mesh(),
      compiler_params=pltpu.CompilerParams(needs_layout_passes=False),
      out_type=(),
      scratch_types=[
          pltpu.VMEM((CHUNK,), jnp.int32),
          pltpu.VMEM((CHUNK,), jnp.int32),
          pltpu.VMEM((CHUNK,), jnp.int32),
          pltpu.VMEM((CHUNK,), jnp.float32),
          pltpu.VMEM((CHUNK,), jnp.float32),
          pltpu.VMEM((CHUNK,), jnp.float32),
          pltpu.VMEM((CHUNK,), jnp.float32),
          pltpu.VMEM((CHUNK,), jnp.float32),
          pltpu.SemaphoreType.DMA,
      ],
  )
  def k6(src_hbm, dst_hbm, expv_hbm, c_hbm, z0_hbm, z1_hbm, out_ref,
         src_v, dst_v, key_v, expv_v, c_v, z0_v, z1_v, outv_v, sem):
    wid = _worker_id()

    @pl.loop(wid, nchunk, step=NW)
    def _chunk(c):
      eb = c * CHUNK
      pltpu.sync_copy(src_hbm.at[pl.ds(eb, CHUNK)], src_v)
      pltpu.sync_copy(dst_hbm.at[pl.ds(eb, CHUNK)], dst_v)
      pltpu.sync_copy(expv_hbm.at[pl.ds(eb, CHUNK)], expv_v)
      pltpu.sync_copy(c_hbm.at[pl.ds(eb, CHUNK)], c_v)
      cp0 = pltpu.async_copy(z0_hbm.at[src_v], z0_v, sem)
      cp1 = pltpu.async_copy(z1_hbm.at[src_v], z1_v, sem)
      cp0.wait()
      cp1.wait()
      for j in range(CHUNK // 16):
        sl = pl.ds(j * 16, 16)
        z = nf + z0_v[sl] + z1_v[sl]
        outv_v[sl] = expv_v[sl] / z + ALPHA * c_v[sl]
        key_v[sl] = src_v[sl] * n + dst_v[sl]
      pltpu.async_copy(outv_v, out_ref.at[key_v], sem).wait()

  return k6


def kernel(inputs, edge, weights, a):
  n, d = inputs.shape
  e = edge.shape[1]
  npad = ((n + NS * 16 - 1) // (NS * 16)) * (NS * 16)

  src = edge[0]
  dst = edge[1]
  avec = a.reshape(-1)

  h = _matmul(inputs, weights)

  s_ids = _make_k2(n, e)(src, dst)
  w_ids, sv, sc = _make_k3(n, e, d)(h, src, dst, avec, s_ids)
  zp, expv, cnt = _make_k4(n, e, npad)(
      w_ids, src, sv[:e], sv[e:], sc[:e], sc[e:])

  zp0 = zp[:npad]
  zp1 = zp[npad:]
  z0 = zp0[:n].reshape(n, 1)
  z1 = zp1[:n].reshape(n, 1)
  filled = _fill(z0, z1, n)

  out_ref = jax.new_ref(filled.reshape(-1))
  _make_k6(n, e)(src, dst, expv, cnt, zp0, zp1, out_ref)
  sgraph = out_ref[...].reshape(n, n)
  return h, sgraph


# speedup vs baseline: 1.9785x; 1.6913x over previous
import functools

import jax
import jax.numpy as jnp
from jax import lax
from jax.experimental import pallas as pl
from jax.experimental.pallas import tpu as pltpu
from jax.experimental.pallas import tpu_sc as plsc

ALPHA = 0.5

NC = 2
NS = 16
NW = NC * NS
CHUNK = 128


def _sc_mesh():
  return plsc.VectorSubcoreMesh(core_axis_name="c", subcore_axis_name="s")


def _worker_id():
  return lax.axis_index("s") * NC + lax.axis_index("c")


def _matmul(x, w):
  n, d_in = x.shape
  d_out = w.shape[1]
  blk = 1000
  grid = n // blk

  def body(x_ref, w_ref, o_ref):
    o_ref[...] = lax.dot_general(
        x_ref[...], w_ref[...], (((1,), (0,)), ((), ())),
        precision=lax.Precision.HIGHEST,
        preferred_element_type=jnp.float32)

  return pl.pallas_call(
      body,
      grid=(grid,),
      in_specs=[
          pl.BlockSpec((blk, d_in), lambda i: (i, 0)),
          pl.BlockSpec((d_in, d_out), lambda i: (0, 0)),
      ],
      out_specs=pl.BlockSpec((blk, d_out), lambda i: (i, 0)),
      out_shape=jax.ShapeDtypeStruct((n, d_out), jnp.float32),
  )(x, w)


def _make_k2(n, e):
  nchunk = e // CHUNK

  @functools.partial(
      pl.kernel,
      mesh=_sc_mesh(),
      compiler_params=pltpu.CompilerParams(needs_layout_passes=False),
      out_type=jax.ShapeDtypeStruct((n * n,), jnp.int32),
      scratch_types=[
          pltpu.VMEM((CHUNK,), jnp.int32),
          pltpu.VMEM((CHUNK,), jnp.int32),
          pltpu.VMEM((CHUNK,), jnp.int32),
          pltpu.VMEM((CHUNK,), jnp.int32),
          pltpu.SemaphoreType.DMA,
      ],
  )
  def k2(src_hbm, dst_hbm, s_hbm, src_v, dst_v, key_v, id_v, sem):
    wid = _worker_id()

    @pl.loop(wid, nchunk, step=NW)
    def _chunk(c):
      eb = c * CHUNK
      pltpu.sync_copy(src_hbm.at[pl.ds(eb, CHUNK)], src_v)
      pltpu.sync_copy(dst_hbm.at[pl.ds(eb, CHUNK)], dst_v)
      for j in range(CHUNK // 16):
        sl = pl.ds(j * 16, 16)
        key_v[sl] = src_v[sl] * n + dst_v[sl]
        id_v[sl] = lax.iota(jnp.int32, 16) + (eb + j * 16)
      pltpu.async_copy(id_v, s_hbm.at[key_v], sem).wait()

  return k2


def _make_k3(n, e, d):
  ch = 64
  nchunk = e // ch
  per_sub = e // NS
  max_chunks = (nchunk + NW - 1) // NW
  npairs = (max_chunks + 1) // 2

  @functools.partial(
      pl.kernel,
      mesh=_sc_mesh(),
      compiler_params=pltpu.CompilerParams(needs_layout_passes=False),
      out_type=(
          jax.ShapeDtypeStruct((e,), jnp.int32),
          jax.ShapeDtypeStruct((2 * e,), jnp.float32),
          jax.ShapeDtypeStruct((2 * e,), jnp.float32),
      ),
      scratch_types=[
          pltpu.VMEM((2, ch), jnp.int32),
          pltpu.VMEM((2, ch), jnp.int32),
          pltpu.VMEM((2, ch), jnp.int32),
          pltpu.VMEM((2, ch), jnp.int32),
          pltpu.VMEM((2, ch, 256), jnp.float32),
          pltpu.VMEM((2, ch, 256), jnp.float32),
          pltpu.VMEM((ch,), jnp.float32),
          pltpu.VMEM((ch,), jnp.float32),
          pltpu.VMEM((272,), jnp.float32),
          pltpu.VMEM((2000,), jnp.float32),
          pltpu.VMEM_SHARED((e,), jnp.float32),
          pltpu.VMEM_SHARED((e,), jnp.float32),
          pltpu.SemaphoreType.DMA,
          pltpu.SemaphoreType.DMA,
          pltpu.SemaphoreType.DMA,
          pltpu.SemaphoreType.DMA,
      ],
  )
  def k3(h_hbm, src_hbm, dst_hbm, a_hbm, s_hbm,
         w_out, sv_out, sc_out,
         src_v, dst_v, key_v, w_v, hs_v, hd_v, val_v, ones_v, a_v, tmp_v,
         sv_sh, sc_sh, seml0, seml1, semg0, semg1):
    cid = lax.axis_index("c")
    sid = lax.axis_index("s")
    wid = sid * NC + cid
    seml = (seml0, seml1)
    semg = (semg0, semg1)

    pltpu.sync_copy(a_hbm, a_v.at[pl.ds(0, d)])
    a_v[pl.ds(d, 16)] = a_v[pl.ds(0, 16)]
    for j in range(ch // 16):
      ones_v[pl.ds(j * 16, 16)] = jnp.full((16,), 1.0, jnp.float32)

    @pl.loop(0, 2000 // 16)
    def _z(i):
      tmp_v[pl.ds(i * 16, 16)] = jnp.zeros((16,), jnp.float32)

    for t in range(per_sub // 2000):
      sl = pl.ds(sid * per_sub + t * 2000, 2000)
      pltpu.sync_copy(tmp_v, sv_sh.at[sl])
      pltpu.sync_copy(tmp_v, sc_sh.at[sl])
    plsc.subcore_barrier()

    def stage_load(c, s):
      @pl.when(c < nchunk)
      def _():
        eb = c * ch
        pltpu.async_copy(src_hbm.at[pl.ds(eb, ch)], src_v.at[s], seml[s])
        pltpu.async_copy(dst_hbm.at[pl.ds(eb, ch)], dst_v.at[s], seml[s])

    def stage_gather(c, s):
      @pl.when(c < nchunk)
      def _():
        pltpu.make_async_copy(
            src_hbm.at[pl.ds(0, ch)], src_v.at[s], seml[s]).wait()
        pltpu.make_async_copy(
            dst_hbm.at[pl.ds(0, ch)], dst_v.at[s], seml[s]).wait()
        for j in range(ch // 16):
          sl = pl.ds(j * 16, 16)
          key_v[s, sl] = src_v[s, sl] * n + dst_v[s, sl]
        pltpu.async_copy(s_hbm.at[key_v.at[s]], w_v.at[s], semg[s])
        pltpu.async_copy(h_hbm.at[src_v.at[s]], hs_v.at[s], semg[s])
        pltpu.async_copy(h_hbm.at[dst_v.at[s]], hd_v.at[s], semg[s])

    def stage_compute(c, s):
      @pl.when(c < nchunk)
      def _():
        pltpu.make_async_copy(
            src_hbm.at[pl.ds(0, ch)], w_v.at[s], semg[s]).wait()
        pltpu.make_async_copy(
            h_hbm.at[pl.ds(0, ch)], hs_v.at[s], semg[s]).wait()
        pltpu.make_async_copy(
            h_hbm.at[pl.ds(0, ch)], hd_v.at[s], semg[s]).wait()
        for g in range(ch // 16):
          e16 = lax.iota(jnp.int32, 16) + g * 16

          @pl.loop(0, d, init_carry=jnp.zeros((16,), jnp.float32), unroll=4)
          def _acc(step, acc):
            col = (lax.iota(jnp.int32, 16) + step) & (d - 1)
            hs = plsc.load_gather(hs_v.at[s], [e16, col])
            hd = plsc.load_gather(hd_v.at[s], [e16, col])
            a16 = a_v[pl.ds(step, 16)]
            return acc + jnp.abs(hs - hd) * a16

          val_v[pl.ds(g * 16, 16)] = jnp.maximum(_acc, 0.0)
        pltpu.sync_copy(val_v, sv_sh.at[w_v.at[s]], add=True)
        pltpu.sync_copy(ones_v, sc_sh.at[w_v.at[s]], add=True)
        pltpu.sync_copy(w_v.at[s], w_out.at[pl.ds(c * ch, ch)])

    stage_load(wid, 0)
    stage_load(wid + NW, 1)

    @pl.loop(0, npairs)
    def _pair(k):
      c0 = wid + (2 * k) * NW
      c1 = c0 + NW
      stage_gather(c0, 0)
      stage_gather(c1, 1)
      stage_compute(c0, 0)
      stage_load(c0 + 2 * NW, 0)
      stage_compute(c1, 1)
      stage_load(c1 + 2 * NW, 1)

    plsc.subcore_barrier()
    for t in range(per_sub // 2000):
      off = sid * per_sub + t * 2000
      sl = pl.ds(off, 2000)
      slo = pl.ds(cid * e + off, 2000)
      pltpu.sync_copy(sv_sh.at[sl], tmp_v)
      pltpu.sync_copy(tmp_v, sv_out.at[slo])
      pltpu.sync_copy(sc_sh.at[sl], tmp_v)
      pltpu.sync_copy(tmp_v, sc_out.at[slo])

  return k3


def _make_k4(n, e, npad):
  nchunk = e // CHUNK
  zslice = npad // NS

  @functools.partial(
      pl.kernel,
      mesh=_sc_mesh(),
      compiler_params=pltpu.CompilerParams(needs_layout_passes=False),
      out_type=(
          jax.ShapeDtypeStruct((2 * npad,), jnp.float32),
          jax.ShapeDtypeStruct((e,), jnp.float32),
          jax.ShapeDtypeStruct((e,), jnp.float32),
      ),
      scratch_types=[
          pltpu.VMEM((CHUNK,), jnp.int32),
          pltpu.VMEM((CHUNK,), jnp.int32),
          pltpu.VMEM((CHUNK,), jnp.float32),
          pltpu.VMEM((CHUNK,), jnp.float32),
          pltpu.VMEM((CHUNK,), jnp.float32),
          pltpu.VMEM((CHUNK,), jnp.float32),
          pltpu.VMEM((CHUNK,), jnp.float32),
          pltpu.VMEM((CHUNK,), jnp.float32),
          pltpu.VMEM((CHUNK,), jnp.float32),
          pltpu.VMEM((zslice,), jnp.float32),
          pltpu.VMEM_SHARED((npad,), jnp.float32),
          pltpu.SemaphoreType.DMA,
      ],
  )
  def k4(w_hbm, src_hbm, sv0_hbm, sv1_hbm, sc0_hbm, sc1_hbm,
         z_out, expv_out, c_out,
         w_v, src_v, t0_v, t1_v, t2_v, t3_v, expv_v, c_v, contrib_v,
         tmp_v, z_sh, sem):
    cid = lax.axis_index("c")
    sid = lax.axis_index("s")
    wid = sid * NC + cid

    @pl.loop(0, zslice // 16)
    def _z(i):
      tmp_v[pl.ds(i * 16, 16)] = jnp.zeros((16,), jnp.float32)

    pltpu.sync_copy(tmp_v, z_sh.at[pl.ds(sid * zslice, zslice)])
    plsc.subcore_barrier()

    @pl.loop(wid, nchunk, step=NW)
    def _chunk(c):
      eb = c * CHUNK
      pltpu.sync_copy(w_hbm.at[pl.ds(eb, CHUNK)], w_v)
      pltpu.sync_copy(src_hbm.at[pl.ds(eb, CHUNK)], src_v)
      cps = [
          pltpu.async_copy(sv0_hbm.at[w_v], t0_v, sem),
          pltpu.async_copy(sv1_hbm.at[w_v], t1_v, sem),
          pltpu.async_copy(sc0_hbm.at[w_v], t2_v, sem),
          pltpu.async_copy(sc1_hbm.at[w_v], t3_v, sem),
      ]
      for cp in cps:
        cp.wait()
      for j in range(CHUNK // 16):
        sl = pl.ds(j * 16, 16)
        val = t0_v[sl] + t1_v[sl]
        ex = jnp.exp(val)
        expv_v[sl] = ex
        c_v[sl] = t2_v[sl] + t3_v[sl]
        eid = lax.iota(jnp.int32, 16) + (eb + j * 16)
        contrib_v[sl] = jnp.where(w_v[sl] == eid, ex - 1.0, 0.0)
      pltpu.sync_copy(contrib_v, z_sh.at[src_v], add=True)
      pltpu.sync_copy(expv_v, expv_out.at[pl.ds(eb, CHUNK)])
      pltpu.sync_copy(c_v, c_out.at[pl.ds(eb, CHUNK)])

    plsc.subcore_barrier()
    pltpu.sync_copy(z_sh.at[pl.ds(sid * zslice, zslice)], tmp_v)
    pltpu.sync_copy(tmp_v, z_out.at[pl.ds(cid * npad + sid * zslice, zslice)])

  return k4


def _fill(z0, z1, n):
  blk = 200
  grid = n // blk
  nf = float(n)

  def body(z0_ref, z1_ref, o_ref):
    z = nf + z0_ref[...] + z1_ref[...]
    o_ref[...] = jnp.broadcast_to(1.0 / z, (blk, n))

  return pl.pallas_call(
      body,
      grid=(grid,),
      in_specs=[
          pl.BlockSpec((blk, 1), lambda i: (i, 0)),
          pl.BlockSpec((blk, 1), lambda i: (i, 0)),
      ],
      out_specs=pl.BlockSpec((blk, n), lambda i: (i, 0)),
      out_shape=jax.ShapeDtypeStruct((n, n), jnp.float32),
  )(z0, z1)


def _make_k6(n, e):
  nchunk = e // CHUNK
  nf = float(n)

  @functools.partial(
      pl.kernel,
      mesh=_sc_mesh(),
      compiler_params=pltpu.CompilerParams(needs_layout_passes=False),
      out_type=(),
      scratch_types=[
          pltpu.VMEM((CHUNK,), jnp.int32),
          pltpu.VMEM((CHUNK,), jnp.int32),
          pltpu.VMEM((CHUNK,), jnp.int32),
          pltpu.VMEM((CHUNK,), jnp.float32),
          pltpu.VMEM((CHUNK,), jnp.float32),
          pltpu.VMEM((CHUNK,), jnp.float32),
          pltpu.VMEM((CHUNK,), jnp.float32),
          pltpu.VMEM((CHUNK,), jnp.float32),
          pltpu.SemaphoreType.DMA,
      ],
  )
  def k6(src_hbm, dst_hbm, expv_hbm, c_hbm, z0_hbm, z1_hbm, out_ref,
         src_v, dst_v, key_v, expv_v, c_v, z0_v, z1_v, outv_v, sem):
    wid = _worker_id()

    @pl.loop(wid, nchunk, step=NW)
    def _chunk(c):
      eb = c * CHUNK
      pltpu.sync_copy(src_hbm.at[pl.ds(eb, CHUNK)], src_v)
      pltpu.sync_copy(dst_hbm.at[pl.ds(eb, CHUNK)], dst_v)
      pltpu.sync_copy(expv_hbm.at[pl.ds(eb, CHUNK)], expv_v)
      pltpu.sync_copy(c_hbm.at[pl.ds(eb, CHUNK)], c_v)
      cp0 = pltpu.async_copy(z0_hbm.at[src_v], z0_v, sem)
      cp1 = pltpu.async_copy(z1_hbm.at[src_v], z1_v, sem)
      cp0.wait()
      cp1.wait()
      for j in range(CHUNK // 16):
        sl = pl.ds(j * 16, 16)
        z = nf + z0_v[sl] + z1_v[sl]
        outv_v[sl] = expv_v[sl] / z + ALPHA * c_v[sl]
        key_v[sl] = src_v[sl] * n + dst_v[sl]
      pltpu.async_copy(outv_v, out_ref.at[key_v], sem).wait()

  return k6


def kernel(inputs, edge, weights, a):
  n, d = inputs.shape
  e = edge.shape[1]
  npad = ((n + NS * 16 - 1) // (NS * 16)) * (NS * 16)

  src = edge[0]
  dst = edge[1]
  avec = a.reshape(-1)

  h = _matmul(inputs, weights)

  s_ids = _make_k2(n, e)(src, dst)
  w_ids, sv, sc = _make_k3(n, e, d)(h, src, dst, avec, s_ids)
  zp, expv, cnt = _make_k4(n, e, npad)(
      w_ids, src, sv[:e], sv[e:], sc[:e], sc[e:])

  zp0 = zp[:npad]
  zp1 = zp[npad:]
  z0 = zp0[:n].reshape(n, 1)
  z1 = zp1[:n].reshape(n, 1)
  filled = _fill(z0, z1, n)

  out_ref = jax.new_ref(filled.reshape(-1))
  _make_k6(n, e)(src, dst, expv, cnt, zp0, zp1, out_ref)
  sgraph = out_ref[...].reshape(n, n)
  return h, sgraph


# speedup vs baseline: 2.5670x; 1.2974x over previous
import functools

import jax
import jax.numpy as jnp
from jax import lax
from jax.experimental import pallas as pl
from jax.experimental.pallas import tpu as pltpu
from jax.experimental.pallas import tpu_sc as plsc

ALPHA = 0.5

NC = 2
NS = 16
NW = NC * NS
CHUNK = 128


def _sc_mesh():
  return plsc.VectorSubcoreMesh(core_axis_name="c", subcore_axis_name="s")


def _worker_id():
  return lax.axis_index("s") * NC + lax.axis_index("c")


def _matmul(x, w):
  n, d_in = x.shape
  d_out = w.shape[1]
  blk = 1000
  grid = n // blk

  def body(x_ref, w_ref, o_ref):
    o_ref[...] = lax.dot_general(
        x_ref[...], w_ref[...], (((1,), (0,)), ((), ())),
        precision=lax.Precision.HIGHEST,
        preferred_element_type=jnp.float32)

  return pl.pallas_call(
      body,
      grid=(grid,),
      in_specs=[
          pl.BlockSpec((blk, d_in), lambda i: (i, 0)),
          pl.BlockSpec((d_in, d_out), lambda i: (0, 0)),
      ],
      out_specs=pl.BlockSpec((blk, d_out), lambda i: (i, 0)),
      out_shape=jax.ShapeDtypeStruct((n, d_out), jnp.float32),
  )(x, w)


def _make_k2(n, e):
  nchunk = e // CHUNK

  @functools.partial(
      pl.kernel,
      mesh=_sc_mesh(),
      compiler_params=pltpu.CompilerParams(needs_layout_passes=False),
      out_type=jax.ShapeDtypeStruct((n * n,), jnp.int32),
      scratch_types=[
          pltpu.VMEM((CHUNK,), jnp.int32),
          pltpu.VMEM((CHUNK,), jnp.int32),
          pltpu.VMEM((CHUNK,), jnp.int32),
          pltpu.VMEM((CHUNK,), jnp.int32),
          pltpu.SemaphoreType.DMA,
      ],
  )
  def k2(src_hbm, dst_hbm, s_hbm, src_v, dst_v, key_v, id_v, sem):
    wid = _worker_id()

    @pl.loop(wid, nchunk, step=NW)
    def _chunk(c):
      eb = c * CHUNK
      pltpu.sync_copy(src_hbm.at[pl.ds(eb, CHUNK)], src_v)
      pltpu.sync_copy(dst_hbm.at[pl.ds(eb, CHUNK)], dst_v)
      for j in range(CHUNK // 16):
        sl = pl.ds(j * 16, 16)
        key_v[sl] = src_v[sl] * n + dst_v[sl]
        id_v[sl] = lax.iota(jnp.int32, 16) + (eb + j * 16)
      pltpu.async_copy(id_v, s_hbm.at[key_v], sem).wait()

  return k2


def _make_k3(n, e, d):
  ch = 64
  nchunk = e // ch
  per_sub = e // NS
  max_chunks = (nchunk + NW - 1) // NW
  npairs = (max_chunks + 1) // 2

  @functools.partial(
      pl.kernel,
      mesh=_sc_mesh(),
      compiler_params=pltpu.CompilerParams(needs_layout_passes=False),
      out_type=(
          jax.ShapeDtypeStruct((e,), jnp.int32),
          jax.ShapeDtypeStruct((2 * e,), jnp.float32),
          jax.ShapeDtypeStruct((2 * e,), jnp.float32),
      ),
      scratch_types=[
          pltpu.VMEM((2, ch), jnp.int32),
          pltpu.VMEM((2, ch), jnp.int32),
          pltpu.VMEM((2, ch), jnp.int32),
          pltpu.VMEM((2, ch), jnp.int32),
          pltpu.VMEM((2, ch, 256), jnp.float32),
          pltpu.VMEM((2, ch, 256), jnp.float32),
          pltpu.VMEM((ch,), jnp.float32),
          pltpu.VMEM((ch,), jnp.float32),
          pltpu.VMEM((272,), jnp.float32),
          pltpu.VMEM((2000,), jnp.float32),
          pltpu.VMEM_SHARED((e,), jnp.float32),
          pltpu.VMEM_SHARED((e,), jnp.float32),
          pltpu.SemaphoreType.DMA,
          pltpu.SemaphoreType.DMA,
          pltpu.SemaphoreType.DMA,
          pltpu.SemaphoreType.DMA,
      ],
  )
  def k3(h_hbm, src_hbm, dst_hbm, a_hbm, s_hbm,
         w_out, sv_out, sc_out,
         src_v, dst_v, key_v, w_v, hs_v, hd_v, val_v, ones_v, a_v, tmp_v,
         sv_sh, sc_sh, seml0, seml1, semg0, semg1):
    cid = lax.axis_index("c")
    sid = lax.axis_index("s")
    wid = sid * NC + cid
    seml = (seml0, seml1)
    semg = (semg0, semg1)

    pltpu.sync_copy(a_hbm, a_v.at[pl.ds(0, d)])
    a_v[pl.ds(d, 16)] = a_v[pl.ds(0, 16)]
    for j in range(ch // 16):
      ones_v[pl.ds(j * 16, 16)] = jnp.full((16,), 1.0, jnp.float32)

    @pl.loop(0, 2000 // 16)
    def _z(i):
      tmp_v[pl.ds(i * 16, 16)] = jnp.zeros((16,), jnp.float32)

    for t in range(per_sub // 2000):
      sl = pl.ds(sid * per_sub + t * 2000, 2000)
      pltpu.sync_copy(tmp_v, sv_sh.at[sl])
      pltpu.sync_copy(tmp_v, sc_sh.at[sl])
    plsc.subcore_barrier()

    def stage_load(c, s):
      @pl.when(c < nchunk)
      def _():
        eb = c * ch
        pltpu.async_copy(src_hbm.at[pl.ds(eb, ch)], src_v.at[s], seml[s])
        pltpu.async_copy(dst_hbm.at[pl.ds(eb, ch)], dst_v.at[s], seml[s])

    def stage_gather(c, s):
      @pl.when(c < nchunk)
      def _():
        pltpu.make_async_copy(
            src_hbm.at[pl.ds(0, ch)], src_v.at[s], seml[s]).wait()
        pltpu.make_async_copy(
            dst_hbm.at[pl.ds(0, ch)], dst_v.at[s], seml[s]).wait()
        for j in range(ch // 16):
          sl = pl.ds(j * 16, 16)
          key_v[s, sl] = src_v[s, sl] * n + dst_v[s, sl]
        pltpu.async_copy(s_hbm.at[key_v.at[s]], w_v.at[s], semg[s])
        pltpu.async_copy(h_hbm.at[src_v.at[s]], hs_v.at[s], semg[s])
        pltpu.async_copy(h_hbm.at[dst_v.at[s]], hd_v.at[s], semg[s])

    def stage_compute(c, s):
      @pl.when(c < nchunk)
      def _():
        pltpu.make_async_copy(
            src_hbm.at[pl.ds(0, ch)], w_v.at[s], semg[s]).wait()
        pltpu.make_async_copy(
            h_hbm.at[pl.ds(0, ch)], hs_v.at[s], semg[s]).wait()
        pltpu.make_async_copy(
            h_hbm.at[pl.ds(0, ch)], hd_v.at[s], semg[s]).wait()
        for g in range(ch // 16):
          e16 = lax.iota(jnp.int32, 16) + g * 16

          @pl.loop(0, d, init_carry=jnp.zeros((16,), jnp.float32), unroll=4)
          def _acc(step, acc):
            col = (lax.iota(jnp.int32, 16) + step) & (d - 1)
            hs = plsc.load_gather(hs_v.at[s], [e16, col])
            hd = plsc.load_gather(hd_v.at[s], [e16, col])
            a16 = a_v[pl.ds(step, 16)]
            return acc + jnp.abs(hs - hd) * a16

          val_v[pl.ds(g * 16, 16)] = jnp.maximum(_acc, 0.0)
        pltpu.sync_copy(val_v, sv_sh.at[w_v.at[s]], add=True)
        pltpu.sync_copy(ones_v, sc_sh.at[w_v.at[s]], add=True)
        pltpu.sync_copy(w_v.at[s], w_out.at[pl.ds(c * ch, ch)])

    stage_load(wid, 0)
    stage_load(wid + NW, 1)

    @pl.loop(0, npairs)
    def _pair(k):
      c0 = wid + (2 * k) * NW
      c1 = c0 + NW
      stage_gather(c0, 0)
      stage_gather(c1, 1)
      stage_compute(c0, 0)
      stage_load(c0 + 2 * NW, 0)
      stage_compute(c1, 1)
      stage_load(c1 + 2 * NW, 1)

    plsc.subcore_barrier()
    for t in range(per_sub // 2000):
      off = sid * per_sub + t * 2000
      sl = pl.ds(off, 2000)
      slo = pl.ds(cid * e + off, 2000)
      pltpu.sync_copy(sv_sh.at[sl], tmp_v)
      pltpu.sync_copy(tmp_v, sv_out.at[slo])
      pltpu.sync_copy(sc_sh.at[sl], tmp_v)
      pltpu.sync_copy(tmp_v, sc_out.at[slo])

  return k3


def _make_k4(n, e, npad):
  nchunk = e // CHUNK
  zslice = npad // NS

  @functools.partial(
      pl.kernel,
      mesh=_sc_mesh(),
      compiler_params=pltpu.CompilerParams(needs_layout_passes=False),
      out_type=(
          jax.ShapeDtypeStruct((2 * npad,), jnp.float32),
          jax.ShapeDtypeStruct((e,), jnp.float32),
          jax.ShapeDtypeStruct((e,), jnp.float32),
      ),
      scratch_types=[
          pltpu.VMEM((CHUNK,), jnp.int32),
          pltpu.VMEM((CHUNK,), jnp.int32),
          pltpu.VMEM((CHUNK,), jnp.float32),
          pltpu.VMEM((CHUNK,), jnp.float32),
          pltpu.VMEM((CHUNK,), jnp.float32),
          pltpu.VMEM((CHUNK,), jnp.float32),
          pltpu.VMEM((CHUNK,), jnp.float32),
          pltpu.VMEM((CHUNK,), jnp.float32),
          pltpu.VMEM((CHUNK,), jnp.float32),
          pltpu.VMEM((zslice,), jnp.float32),
          pltpu.VMEM_SHARED((npad,), jnp.float32),
          pltpu.SemaphoreType.DMA,
      ],
  )
  def k4(w_hbm, src_hbm, sv0_hbm, sv1_hbm, sc0_hbm, sc1_hbm,
         z_out, expv_out, c_out,
         w_v, src_v, t0_v, t1_v, t2_v, t3_v, expv_v, c_v, contrib_v,
         tmp_v, z_sh, sem):
    cid = lax.axis_index("c")
    sid = lax.axis_index("s")
    wid = sid * NC + cid

    @pl.loop(0, zslice // 16)
    def _z(i):
      tmp_v[pl.ds(i * 16, 16)] = jnp.zeros((16,), jnp.float32)

    pltpu.sync_copy(tmp_v, z_sh.at[pl.ds(sid * zslice, zslice)])
    plsc.subcore_barrier()

    @pl.loop(wid, nchunk, step=NW)
    def _chunk(c):
      eb = c * CHUNK
      pltpu.sync_copy(w_hbm.at[pl.ds(eb, CHUNK)], w_v)
      pltpu.sync_copy(src_hbm.at[pl.ds(eb, CHUNK)], src_v)
      cps = [
          pltpu.async_copy(sv0_hbm.at[w_v], t0_v, sem),
          pltpu.async_copy(sv1_hbm.at[w_v], t1_v, sem),
          pltpu.async_copy(sc0_hbm.at[w_v], t2_v, sem),
          pltpu.async_copy(sc1_hbm.at[w_v], t3_v, sem),
      ]
      for cp in cps:
        cp.wait()
      for j in range(CHUNK // 16):
        sl = pl.ds(j * 16, 16)
        val = t0_v[sl] + t1_v[sl]
        ex = jnp.exp(val)
        expv_v[sl] = ex
        c_v[sl] = t2_v[sl] + t3_v[sl]
        eid = lax.iota(jnp.int32, 16) + (eb + j * 16)
        contrib_v[sl] = jnp.where(w_v[sl] == eid, ex - 1.0, 0.0)
      pltpu.sync_copy(contrib_v, z_sh.at[src_v], add=True)
      pltpu.sync_copy(expv_v, expv_out.at[pl.ds(eb, CHUNK)])
      pltpu.sync_copy(c_v, c_out.at[pl.ds(eb, CHUNK)])

    plsc.subcore_barrier()
    pltpu.sync_copy(z_sh.at[pl.ds(sid * zslice, zslice)], tmp_v)
    pltpu.sync_copy(tmp_v, z_out.at[pl.ds(cid * npad + sid * zslice, zslice)])

  return k4


def _make_k5(n, npad):
  rows_per = 320
  rb = 4
  nf = float(n)

  @functools.partial(
      pl.kernel,
      mesh=_sc_mesh(),
      compiler_params=pltpu.CompilerParams(needs_layout_passes=False),
      out_type=jax.ShapeDtypeStruct((n * n,), jnp.float32),
      scratch_types=[
          pltpu.VMEM((rows_per,), jnp.float32),
          pltpu.VMEM((rows_per,), jnp.float32),
          pltpu.VMEM((rows_per + 32,), jnp.float32),
          pltpu.VMEM((rb * n,), jnp.float32),
          pltpu.VMEM((rb * n,), jnp.float32),
          pltpu.SemaphoreType.DMA,
          pltpu.SemaphoreType.DMA,
      ],
  )
  def k5(z0_hbm, z1_hbm, out_hbm, z0w, z1w, izw, rowbuf0, rowbuf1, sem0,
         sem1):
    wid = _worker_id()
    r0 = wid * rows_per
    pltpu.sync_copy(z0_hbm.at[pl.ds(r0, rows_per)], z0w)
    pltpu.sync_copy(z1_hbm.at[pl.ds(r0, rows_per)], z1w)
    for j in range(rows_per // 16):
      sl = pl.ds(j * 16, 16)
      izw[sl] = 1.0 / (nf + z0w[sl] + z1w[sl])
    nrows = jnp.minimum(rows_per, n - r0)
    nblk = nrows // rb
    sems = (sem0, sem1)
    rowbufs = (rowbuf0, rowbuf1)

    def do_block(b, s):
      @pl.when(b < nblk)
      def _():
        @pl.when(b >= 2)
        def _():
          pltpu.make_async_copy(
              out_hbm.at[pl.ds(0, rb * n)], rowbufs[s], sems[s]).wait()
        iz16 = izw[pl.ds(b * rb, 16)]
        for t in range(rb):
          val = jnp.full((16,), iz16[t], jnp.float32)

          @pl.loop(0, n // 16, unroll=16)
          def _st(i):
            rowbufs[s][pl.ds(t * n + i * 16, 16)] = val

        pltpu.async_copy(
            rowbufs[s], out_hbm.at[pl.ds((r0 + b * rb) * n, rb * n)],
            sems[s])

    @pl.loop(0, rows_per // rb // 2)
    def _pair(k):
      do_block(2 * k, 0)
      do_block(2 * k + 1, 1)

    pltpu.make_async_copy(
        out_hbm.at[pl.ds(0, rb * n)], rowbuf0, sem0).wait()
    pltpu.make_async_copy(
        out_hbm.at[pl.ds(0, rb * n)], rowbuf1, sem1).wait()

  return k5


def _make_k6(n, e):
  nchunk = e // CHUNK
  nf = float(n)

  @functools.partial(
      pl.kernel,
      mesh=_sc_mesh(),
      compiler_params=pltpu.CompilerParams(needs_layout_passes=False),
      out_type=(),
      scratch_types=[
          pltpu.VMEM((CHUNK,), jnp.int32),
          pltpu.VMEM((CHUNK,), jnp.int32),
          pltpu.VMEM((CHUNK,), jnp.int32),
          pltpu.VMEM((CHUNK,), jnp.float32),
          pltpu.VMEM((CHUNK,), jnp.float32),
          pltpu.VMEM((CHUNK,), jnp.float32),
          pltpu.VMEM((CHUNK,), jnp.float32),
          pltpu.VMEM((CHUNK,), jnp.float32),
          pltpu.SemaphoreType.DMA,
      ],
  )
  def k6(src_hbm, dst_hbm, expv_hbm, c_hbm, z0_hbm, z1_hbm, out_ref,
         src_v, dst_v, key_v, expv_v, c_v, z0_v, z1_v, outv_v, sem):
    wid = _worker_id()

    @pl.loop(wid, nchunk, step=NW)
    def _chunk(c):
      eb = c * CHUNK
      pltpu.sync_copy(src_hbm.at[pl.ds(eb, CHUNK)], src_v)
      pltpu.sync_copy(dst_hbm.at[pl.ds(eb, CHUNK)], dst_v)
      pltpu.sync_copy(expv_hbm.at[pl.ds(eb, CHUNK)], expv_v)
      pltpu.sync_copy(c_hbm.at[pl.ds(eb, CHUNK)], c_v)
      cp0 = pltpu.async_copy(z0_hbm.at[src_v], z0_v, sem)
      cp1 = pltpu.async_copy(z1_hbm.at[src_v], z1_v, sem)
      cp0.wait()
      cp1.wait()
      for j in range(CHUNK // 16):
        sl = pl.ds(j * 16, 16)
        z = nf + z0_v[sl] + z1_v[sl]
        outv_v[sl] = expv_v[sl] / z + ALPHA * c_v[sl]
        key_v[sl] = src_v[sl] * n + dst_v[sl]
      pltpu.async_copy(outv_v, out_ref.at[key_v], sem).wait()

  return k6


def kernel(inputs, edge, weights, a):
  n, d = inputs.shape
  e = edge.shape[1]
  npad = ((n + NS * 16 - 1) // (NS * 16)) * (NS * 16)

  src = edge[0]
  dst = edge[1]
  avec = a.reshape(-1)

  h = _matmul(inputs, weights)

  s_ids = _make_k2(n, e)(src, dst)
  w_ids, sv, sc = _make_k3(n, e, d)(h, src, dst, avec, s_ids)
  zp, expv, cnt = _make_k4(n, e, npad)(
      w_ids, src, sv[:e], sv[e:], sc[:e], sc[e:])

  zp0 = zp[:npad]
  zp1 = zp[npad:]
  filled = _make_k5(n, npad)(zp0, zp1)

  out_ref = jax.new_ref(filled)
  _make_k6(n, e)(src, dst, expv, cnt, zp0, zp1, out_ref)
  sgraph = out_ref[...].reshape(n, n)
  return h, sgraph
